# baseline (speedup 1.0000x reference)
import jax
import jax.numpy as jnp
from jax import lax
from jax.experimental import pallas as pl
from jax.experimental.pallas import tpu as pltpu
from jax.experimental.pallas import tpu_sc as plsc

V = 1000000
D = 64
B = 16384
C = 20
NEG = 3
CN = C + NEG
NC = 2
NS = 16
NW = NC * NS
BPW = B // NW
CHUNK = 2
NSTEPS = BPW // CHUNK


def _gather_body(cidx_hbm, ctx_hbm, tgt_sel_hbm, out_hbm,
                 cidx_v,
                 ctx_rows0, ctx_rows1, tgt_rows0, tgt_rows1,
                 out_v0, out_v1, sem_c0, sem_c1, sem_t0, sem_t1):
    cid = lax.axis_index("c")
    sid = lax.axis_index("s")
    wid = sid * NC + cid
    base = wid * BPW

    pltpu.sync_copy(cidx_hbm.at[pl.ds(base * CN, BPW * CN)],
                    cidx_v.at[pl.ds(0, BPW * CN)])

    ctx_rows = (ctx_rows0, ctx_rows1)
    tgt_rows = (tgt_rows0, tgt_rows1)
    outs = (out_v0, out_v1)
    sems_c = (sem_c0, sem_c1)
    sems_t = (sem_t0, sem_t1)

    def issue(s, b):
        s0 = s * (CHUNK * CN)

        def issue_row(j, carry2):
            r = cidx_v[pl.ds(s0 + j, 16)][0]
            r0 = pl.multiple_of((r >> 3) * 8, 8)
            pltpu.async_copy(ctx_hbm.at[pl.ds(r0, 8), :],
                             ctx_rows[b].at[j], sems_c[b])
            return carry2

        lax.fori_loop(0, CHUNK * CN, issue_row, 0, unroll=False)
        pltpu.async_copy(tgt_sel_hbm.at[pl.ds(base + s * CHUNK, CHUNK), :],
                         tgt_rows[b], sems_t[b])

    def wait_bufs(b):
        pltpu.make_async_copy(ctx_hbm.at[pl.ds(0, CHUNK * CN * 8), :],
                              ctx_rows[b].bitcast(jnp.float32), sems_c[b]).wait()
        pltpu.make_async_copy(tgt_sel_hbm.at[pl.ds(0, CHUNK), :],
                              tgt_rows[b], sems_t[b]).wait()

    def step_b(s, b):
        ib = base + s * CHUNK

        @pl.when(s + 1 < NSTEPS)
        def _():
            issue(s + 1, 1 - b)

        wait_bufs(b)
        crv = ctx_rows[b]
        trv = tgt_rows[b]
        ov = outs[b]

        def item(i, carry2):
            ib23 = i * CN
            t0 = trv[i, pl.ds(0, 16)]
            t1 = trv[i, pl.ds(16, 16)]
            t2 = trv[i, pl.ds(32, 16)]
            t3 = trv[i, pl.ds(48, 16)]
            cs0 = jnp.zeros((16,), jnp.float32)
            cs1 = jnp.zeros((16,), jnp.float32)
            cs2 = jnp.zeros((16,), jnp.float32)
            cs3 = jnp.zeros((16,), jnp.float32)
            gi23 = s * (CHUNK * CN) + ib23
            for j in range(C):
                su = cidx_v[pl.ds(gi23 + j, 16)][0] & 7
                cs0 = cs0 + crv[ib23 + j, su, pl.ds(0, 16)]
                cs1 = cs1 + crv[ib23 + j, su, pl.ds(16, 16)]
                cs2 = cs2 + crv[ib23 + j, su, pl.ds(32, 16)]
                cs3 = cs3 + crv[ib23 + j, su, pl.ds(48, 16)]
            ns0 = jnp.zeros((16,), jnp.float32)
            ns1 = jnp.zeros((16,), jnp.float32)
            ns2 = jnp.zeros((16,), jnp.float32)
            ns3 = jnp.zeros((16,), jnp.float32)
            for j in range(C, CN):
                su = cidx_v[pl.ds(gi23 + j, 16)][0] & 7
                ns0 = ns0 + crv[ib23 + j, su, pl.ds(0, 16)]
                ns1 = ns1 + crv[ib23 + j, su, pl.ds(16, 16)]
                ns2 = ns2 + crv[ib23 + j, su, pl.ds(32, 16)]
                ns3 = ns3 + crv[ib23 + j, su, pl.ds(48, 16)]
            pacc = cs0 * t0 + cs1 * t1 + cs2 * t2 + cs3 * t3
            nacc = ns0 * t0 + ns1 * t1 + ns2 * t2 + ns3 * t3
            ov[i, pl.ds(0, 16)] = pacc
            ov[i, pl.ds(16, 16)] = nacc
            return carry2

        lax.fori_loop(0, CHUNK, item, 0, unroll=False)
        pltpu.sync_copy(ov, out_hbm.at[pl.ds(ib, CHUNK)])

    issue(0, 0)

    def step(s, carry):
        b = lax.rem(s, 2)

        @pl.when(b == 0)
        def _():
            step_b(s, 0)

        @pl.when(b == 1)
        def _():
            step_b(s, 1)

        return carry

    lax.fori_loop(0, NSTEPS, step, 0, unroll=False)


def _tc_body(part_ref, out_ref):
    x = part_ref[...]
    p = jnp.sum(x[:, :16], axis=1) * (1.0 / C)
    n = -jnp.sum(x[:, 16:], axis=1)

    def logsig(v):
        return jnp.minimum(v, 0.0) - jnp.log1p(jnp.exp(-jnp.abs(v)))

    total = jnp.sum(logsig(p) + logsig(n))
    out_ref[0, 0] = -total * (1.0 / B)


def kernel(targets, contexts, negsamples, context_emb, target_emb):
    cidx = jnp.concatenate(
        [contexts.astype(jnp.int32), negsamples.astype(jnp.int32)],
        axis=1).reshape(B * CN)
    tgt_sel = jnp.take(target_emb, targets, axis=0)

    mesh = plsc.VectorSubcoreMesh(core_axis_name="c", subcore_axis_name="s",
                                  num_cores=NC, num_subcores=NS)
    gather = pl.kernel(
        _gather_body,
        out_type=jax.ShapeDtypeStruct((B, 32), jnp.float32),
        mesh=mesh,
        scratch_types=[
            pltpu.VMEM((BPW * CN + 16,), jnp.int32),
            pltpu.VMEM((CHUNK * CN, 8, D), jnp.float32),
            pltpu.VMEM((CHUNK * CN, 8, D), jnp.float32),
            pltpu.VMEM((CHUNK, D), jnp.float32),
            pltpu.VMEM((CHUNK, D), jnp.float32),
            pltpu.VMEM((CHUNK, 32), jnp.float32),
            pltpu.VMEM((CHUNK, 32), jnp.float32),
            pltpu.SemaphoreType.DMA,
            pltpu.SemaphoreType.DMA,
            pltpu.SemaphoreType.DMA,
            pltpu.SemaphoreType.DMA,
        ],
    )
    part = gather(cidx, context_emb, tgt_sel)

    loss = pl.pallas_call(
        _tc_body,
        out_shape=jax.ShapeDtypeStruct((1, 1), jnp.float32),
        in_specs=[pl.BlockSpec(memory_space=pltpu.VMEM)],
        out_specs=pl.BlockSpec(memory_space=pltpu.SMEM),
    )(part)
    return loss


# speedup vs baseline: 1.3372x; 1.3372x over previous
import jax
import jax.numpy as jnp
from jax import lax
from jax.experimental import pallas as pl
from jax.experimental.pallas import tpu as pltpu
from jax.experimental.pallas import tpu_sc as plsc

V = 1000000
D = 64
B = 16384
C = 20
NEG = 3
CN = C + NEG
NC = 2
NS = 16
NW = NC * NS
BPW = B // NW
CHUNK = 16
NSTEPS = BPW // CHUNK


def _gather_body(cidx_hbm, ctx_hbm, tgt_sel_hbm, out_hbm,
                 cidx_v,
                 ctx_rows0, ctx_rows1, tgt_rows0, tgt_rows1,
                 out_v0, out_v1, sem_c0, sem_c1, sem_t0, sem_t1):
    cid = lax.axis_index("c")
    sid = lax.axis_index("s")
    wid = sid * NC + cid
    base = wid * BPW

    pltpu.sync_copy(cidx_hbm.at[pl.ds(base * CN, BPW * CN)], cidx_v)

    ctx_rows = (ctx_rows0, ctx_rows1)
    tgt_rows = (tgt_rows0, tgt_rows1)
    outs = (out_v0, out_v1)
    sems_c = (sem_c0, sem_c1)
    sems_t = (sem_t0, sem_t1)

    def issue(s, b):
        pltpu.async_copy(
            ctx_hbm.at[cidx_v.at[pl.ds(s * (CHUNK * CN), CHUNK * CN)]],
            ctx_rows[b], sems_c[b])
        pltpu.async_copy(tgt_sel_hbm.at[pl.ds(base + s * CHUNK, CHUNK), :],
                         tgt_rows[b], sems_t[b])

    def wait_bufs(b):
        pltpu.make_async_copy(
            ctx_hbm.at[cidx_v.at[pl.ds(0, CHUNK * CN)]],
            ctx_rows[b], sems_c[b]).wait()
        pltpu.make_async_copy(tgt_sel_hbm.at[pl.ds(0, CHUNK), :],
                              tgt_rows[b], sems_t[b]).wait()

    def step_b(s, b):
        ib = base + s * CHUNK

        @pl.when(s + 1 < NSTEPS)
        def _():
            issue(s + 1, 1 - b)

        wait_bufs(b)
        crv = ctx_rows[b]
        trv = tgt_rows[b]
        ov = outs[b]

        def item(i, carry2):
            ib23 = i * CN
            t0 = trv[i, pl.ds(0, 16)]
            t1 = trv[i, pl.ds(16, 16)]
            t2 = trv[i, pl.ds(32, 16)]
            t3 = trv[i, pl.ds(48, 16)]
            cs0 = jnp.zeros((16,), jnp.float32)
            cs1 = jnp.zeros((16,), jnp.float32)
            cs2 = jnp.zeros((16,), jnp.float32)
            cs3 = jnp.zeros((16,), jnp.float32)
            for j in range(C):
                cs0 = cs0 + crv[ib23 + j, pl.ds(0, 16)]
                cs1 = cs1 + crv[ib23 + j, pl.ds(16, 16)]
                cs2 = cs2 + crv[ib23 + j, pl.ds(32, 16)]
                cs3 = cs3 + crv[ib23 + j, pl.ds(48, 16)]
            ns0 = jnp.zeros((16,), jnp.float32)
            ns1 = jnp.zeros((16,), jnp.float32)
            ns2 = jnp.zeros((16,), jnp.float32)
            ns3 = jnp.zeros((16,), jnp.float32)
            for j in range(C, CN):
                ns0 = ns0 + crv[ib23 + j, pl.ds(0, 16)]
                ns1 = ns1 + crv[ib23 + j, pl.ds(16, 16)]
                ns2 = ns2 + crv[ib23 + j, pl.ds(32, 16)]
                ns3 = ns3 + crv[ib23 + j, pl.ds(48, 16)]
            pacc = cs0 * t0 + cs1 * t1 + cs2 * t2 + cs3 * t3
            nacc = ns0 * t0 + ns1 * t1 + ns2 * t2 + ns3 * t3
            ov[i, pl.ds(0, 16)] = pacc
            ov[i, pl.ds(16, 16)] = nacc
            return carry2

        lax.fori_loop(0, CHUNK, item, 0, unroll=False)
        pltpu.sync_copy(ov, out_hbm.at[pl.ds(ib, CHUNK)])

    issue(0, 0)

    def step(s, carry):
        b = lax.rem(s, 2)

        @pl.when(b == 0)
        def _():
            step_b(s, 0)

        @pl.when(b == 1)
        def _():
            step_b(s, 1)

        return carry

    lax.fori_loop(0, NSTEPS, step, 0, unroll=False)


def _tc_body(part_ref, out_ref):
    x = part_ref[...]
    p = jnp.sum(x[:, :16], axis=1) * (1.0 / C)
    n = -jnp.sum(x[:, 16:], axis=1)

    def logsig(v):
        return jnp.minimum(v, 0.0) - jnp.log1p(jnp.exp(-jnp.abs(v)))

    total = jnp.sum(logsig(p) + logsig(n))
    out_ref[0, 0] = -total * (1.0 / B)


def kernel(targets, contexts, negsamples, context_emb, target_emb):
    cidx = jnp.concatenate(
        [contexts.astype(jnp.int32), negsamples.astype(jnp.int32)],
        axis=1).reshape(B * CN)
    tgt_sel = jnp.take(target_emb, targets, axis=0)

    mesh = plsc.VectorSubcoreMesh(core_axis_name="c", subcore_axis_name="s",
                                  num_cores=NC, num_subcores=NS)
    gather = pl.kernel(
        _gather_body,
        out_type=jax.ShapeDtypeStruct((B, 32), jnp.float32),
        mesh=mesh,
        compiler_params=pltpu.CompilerParams(use_tc_tiling_on_sc=False),
        scratch_types=[
            pltpu.VMEM((BPW * CN,), jnp.int32),
            pltpu.VMEM((CHUNK * CN, D), jnp.float32),
            pltpu.VMEM((CHUNK * CN, D), jnp.float32),
            pltpu.VMEM((CHUNK, D), jnp.float32),
            pltpu.VMEM((CHUNK, D), jnp.float32),
            pltpu.VMEM((CHUNK, 32), jnp.float32),
            pltpu.VMEM((CHUNK, 32), jnp.float32),
            pltpu.SemaphoreType.DMA,
            pltpu.SemaphoreType.DMA,
            pltpu.SemaphoreType.DMA,
            pltpu.SemaphoreType.DMA,
        ],
    )
    part = gather(cidx, context_emb, tgt_sel)

    loss = pl.pallas_call(
        _tc_body,
        out_shape=jax.ShapeDtypeStruct((1, 1), jnp.float32),
        in_specs=[pl.BlockSpec(memory_space=pltpu.VMEM)],
        out_specs=pl.BlockSpec(memory_space=pltpu.SMEM),
    )(part)
    return loss
